# Initial kernel scaffold; baseline (speedup 1.0000x reference)
import jax
import jax.numpy as jnp
from jax.experimental import pallas as pl


def kernel(tc, embedding):
    raise NotImplementedError("write your pallas kernel here")


# speedup vs baseline: 3.5004x; 3.5004x over previous
import jax
import jax.numpy as jnp
from jax.experimental import pallas as pl
from jax.experimental.pallas import tpu as pltpu
from jax.experimental.pallas import tpu_sc as plsc

_WINDOW = 128


def kernel(tc, embedding):
    batch, hist = tc.shape
    n_rows, depth = embedding.shape
    n_idx = batch * hist
    n_pairs = n_idx // 2

    left = jnp.broadcast_to(embedding[:, None, :], (n_rows, n_rows, depth))
    right = jnp.broadcast_to(embedding[None, :, :], (n_rows, n_rows, depth))
    pair_table = jnp.concatenate([left, right], axis=-1).reshape(
        n_rows * n_rows, 2 * depth)

    flat = tc.reshape(n_pairs, 2)
    pair_idx = (flat[:, 0] * n_rows + flat[:, 1]).reshape(1, n_pairs)

    mesh = plsc.VectorSubcoreMesh(core_axis_name="core",
                                  subcore_axis_name="subcore")

    @pl.kernel(
        out_type=jax.ShapeDtypeStruct((n_pairs, 2 * depth), embedding.dtype),
        mesh=mesh,
    )
    def gather_kernel(table_hbm, i_hbm, o_hbm):
        def body(i_vmem, o_vmem):
            pltpu.sync_copy(table_hbm.at[i_vmem.at[0]], o_vmem)

        pltpu.emit_pipeline(
            body,
            grid=(n_pairs // _WINDOW,),
            in_specs=[pl.BlockSpec((1, _WINDOW), lambda i: (0, i))],
            out_specs=[pl.BlockSpec((_WINDOW, 2 * depth), lambda i: (i, 0))],
            core_axis_name=("core", "subcore"),
            dimension_semantics=(pltpu.PARALLEL,),
        )(i_hbm, o_hbm)

    out = gather_kernel(pair_table, pair_idx)
    return out.reshape(batch, hist, depth)


# speedup vs baseline: 3.6386x; 1.0395x over previous
import jax
import jax.numpy as jnp
from jax import lax
from jax.experimental import pallas as pl
from jax.experimental.pallas import tpu as pltpu
from jax.experimental.pallas import tpu_sc as plsc

_NC = 2
_NS = 16
_NW = _NC * _NS
_CH = 128
_NBUF = 4


def kernel(tc, embedding):
    batch, hist = tc.shape
    n_rows, depth = embedding.shape
    width = 2 * depth
    n_idx = batch * hist
    n_pairs = n_idx // 2
    per_worker = n_pairs // _NW
    n_chunks = per_worker // _CH
    assert per_worker % _CH == 0 and (n_chunks - _NBUF) % _NBUF == 0

    left = jnp.broadcast_to(embedding[:, None, :], (n_rows, n_rows, depth))
    right = jnp.broadcast_to(embedding[None, :, :], (n_rows, n_rows, depth))
    pair_table = jnp.concatenate([left, right], axis=-1).reshape(
        n_rows * n_rows, width)

    flat = tc.reshape(n_pairs, 2)
    pair_idx = flat[:, 0] * n_rows + flat[:, 1]

    mesh = plsc.VectorSubcoreMesh(core_axis_name="core",
                                  subcore_axis_name="subcore")

    scratch = (
        [pltpu.VMEM((_CH,), jnp.int32) for _ in range(_NBUF)]
        + [pltpu.VMEM((_CH, width), jnp.float32) for _ in range(_NBUF)]
        + [pltpu.SemaphoreType.DMA for _ in range(2 * _NBUF)]
    )

    @pl.kernel(
        out_type=jax.ShapeDtypeStruct((n_pairs, width), embedding.dtype),
        mesh=mesh,
        scratch_types=scratch,
    )
    def gather_kernel(table_hbm, i_hbm, o_hbm, *bufs):
        idx_v = bufs[:_NBUF]
        rows_v = bufs[_NBUF:2 * _NBUF]
        gsem = bufs[2 * _NBUF:3 * _NBUF]
        ssem = bufs[3 * _NBUF:]

        wid = lax.axis_index("subcore") * _NC + lax.axis_index("core")
        base = wid * per_worker

        def load_and_gather(b, k):
            pltpu.sync_copy(i_hbm.at[pl.ds(base + k * _CH, _CH)], idx_v[b])
            pltpu.make_async_copy(
                table_hbm.at[idx_v[b]], rows_v[b], gsem[b]).start()

        def complete(b, k):
            pltpu.make_async_copy(
                table_hbm.at[idx_v[b]], rows_v[b], gsem[b]).wait()
            store = pltpu.make_async_copy(
                rows_v[b], o_hbm.at[pl.ds(base + k * _CH, _CH)], ssem[b])
            store.start()
            store.wait()

        for b in range(_NBUF):
            load_and_gather(b, b)

        @pl.loop(_NBUF, n_chunks, step=_NBUF)
        def _(k0):
            for b in range(_NBUF):
                complete(b, k0 + b - _NBUF)
                load_and_gather(b, k0 + b)

        for b in range(_NBUF):
            complete(b, n_chunks - _NBUF + b)

    out = gather_kernel(pair_table, pair_idx)
    return out.reshape(batch, hist, depth)
